# baseline (speedup 1.0000x reference)
import jax
import jax.numpy as jnp
from jax import lax
from jax.experimental import pallas as pl
from jax.experimental.pallas import tpu as pltpu
from jax.experimental.pallas import tpu_sc as plsc

N_NODES = 10000
D = 128
N_EDGES = 320000

_NC = 2
_NS = 16
_NW = _NC * _NS
_EPW = N_EDGES // _NW
_K = 80
_NCHUNK = _EPW // _K
_RPT = 624
_TAIL0 = _NS * _RPT
_TAIL = N_NODES - _TAIL0


def _sc_body(h_hbm, ei_hbm, out_hbm, idx_s, idx_d, rows, acc, gsem, ssem):
    c = lax.axis_index("c")
    s = lax.axis_index("s")
    w = s * _NC + c
    row0 = s * _RPT

    isrc = pltpu.async_copy(ei_hbm.at[pl.ds(w * _EPW, _EPW)], idx_s, gsem[0])
    idst = pltpu.async_copy(ei_hbm.at[pl.ds(N_EDGES + w * _EPW, _EPW)],
                            idx_d, gsem[1])

    def _gather(i, b):
        pltpu.async_copy(h_hbm.at[idx_s.at[pl.ds(i * _K, _K)]], rows[b], gsem[b])

    def _gwait(i, b):
        pltpu.make_async_copy(h_hbm.at[idx_s.at[pl.ds(i * _K, _K)]],
                              rows[b], gsem[b]).wait()

    def _scatter(i, b):
        pltpu.async_copy(rows[b], acc.at[idx_d.at[pl.ds(i * _K, _K)]],
                         ssem[b], add=True)

    def _swait(i, b):
        pltpu.make_async_copy(rows[b], acc.at[idx_d.at[pl.ds(i * _K, _K)]],
                              ssem[b]).wait()

    z = jnp.zeros((16,), jnp.float32)

    def _zfill(i, _):
        for j in range(D // 16):
            rows[0][i, pl.ds(j * 16, 16)] = z
        return 0

    lax.fori_loop(0, _K, _zfill, 0)
    isrc.wait()
    idst.wait()
    for t in range(_RPT // _K):
        pltpu.sync_copy(rows[0], acc.at[pl.ds(row0 + t * _K, _K)])
    pltpu.sync_copy(rows[0].at[pl.ds(0, _RPT % _K)],
                    acc.at[pl.ds(row0 + (_RPT // _K) * _K, _RPT % _K)])

    @pl.when(s == 0)
    def _zero_tail():
        pltpu.sync_copy(rows[0].at[pl.ds(0, _TAIL)],
                        acc.at[pl.ds(_TAIL0, _TAIL)])

    plsc.subcore_barrier()

    _gather(0, 0)
    _gather(1, 1)
    _gwait(0, 0)
    _scatter(0, 0)
    _gather(2, 2)
    _gwait(1, 1)
    _scatter(1, 1)
    _swait(0, 0)
    _gather(3, 0)

    def _step(i, b):
        _gwait(i, b)
        _scatter(i, b)
        _swait(i - 1, (b + 2) % 3)
        _gather(i + 2, (b + 2) % 3)

    def _body(j, _):
        i = 3 * j + 2
        _step(i, 2)
        _step(i + 1, 0)
        _step(i + 2, 1)
        return 0

    _L = (_NCHUNK - 4) // 3
    lax.fori_loop(0, _L, _body, 0)
    for i in range(2 + 3 * _L, _NCHUNK):
        _gwait(i, i % 3)
        _scatter(i, i % 3)
        _swait(i - 1, (i - 1) % 3)
        if i + 2 < _NCHUNK:
            _gather(i + 2, (i + 2) % 3)
    _swait(_NCHUNK - 1, (_NCHUNK - 1) % 3)
    plsc.subcore_barrier()
    pltpu.sync_copy(acc.at[pl.ds(row0, _RPT)], out_hbm.at[c, pl.ds(row0, _RPT)])

    @pl.when(s == 0)
    def _copy_tail():
        pltpu.sync_copy(acc.at[pl.ds(_TAIL0, _TAIL)],
                        out_hbm.at[c, pl.ds(_TAIL0, _TAIL)])


def _sc_segment_sum(h, ei):
    mesh = plsc.VectorSubcoreMesh(core_axis_name="c", subcore_axis_name="s")
    f = pl.kernel(
        _sc_body,
        out_type=jax.ShapeDtypeStruct((_NC, N_NODES, D), jnp.float32),
        mesh=mesh,
        scratch_types=[
            pltpu.VMEM((_EPW,), jnp.int32),
            pltpu.VMEM((_EPW,), jnp.int32),
            [pltpu.VMEM((_K, D), jnp.float32) for _ in range(3)],
            pltpu.VMEM_SHARED((N_NODES, D), jnp.float32),
            [pltpu.SemaphoreType.DMA for _ in range(3)],
            [pltpu.SemaphoreType.DMA for _ in range(3)],
        ],
    )
    return f(h, ei)


_BLK = 1000


def _pre_body(x_ref, w_ref, b_ref, p_ref):
    p_ref[...] = jnp.dot(x_ref[...], w_ref[...]) + b_ref[...]


def _mid_body(h_ref, w_ref, b_ref, wl_ref, bl_ref, p_ref, part_ref):
    h = h_ref[...]
    p_ref[...] = jnp.dot(h, w_ref[...]) + b_ref[...]
    part_ref[...] = jnp.dot(h, wl_ref[...]) + bl_ref[...]


def _l0_body(p_ref, a_ref, w1_ref, w2_ref, b2_ref, h_ref):
    t = jnp.maximum(p_ref[...] + jnp.dot(a_ref[0] + a_ref[1], w1_ref[...]), 0.0)
    h_ref[...] = jnp.maximum(jnp.dot(t, w2_ref[...]) + b2_ref[...], 0.0)


def _l1_body(p_ref, a_ref, w1_ref, w2_ref, b2_ref, wl_ref, part_ref, out_ref):
    t = jnp.maximum(p_ref[...] + jnp.dot(a_ref[0] + a_ref[1], w1_ref[...]), 0.0)
    h2 = jnp.maximum(jnp.dot(t, w2_ref[...]) + b2_ref[...], 0.0)
    logits = part_ref[...] + jnp.dot(h2, wl_ref[...])
    m = jnp.max(logits, axis=1, keepdims=True)
    lse = jnp.log(jnp.sum(jnp.exp(logits - m), axis=1, keepdims=True)) + m
    out_ref[...] = logits - lse


def _row_spec():
    return pl.BlockSpec((_BLK, D), lambda i: (i, 0))


def _agg_spec():
    return pl.BlockSpec((_NC, _BLK, D), lambda i: (0, i, 0))


def _full_spec(r, c):
    return pl.BlockSpec((r, c), lambda i: (0, 0))


_GRID = (N_NODES // _BLK,)
_F32 = jax.ShapeDtypeStruct((N_NODES, D), jnp.float32)


def kernel(x, edge_index, W01, b01, W02, b02, W11, b11, W12, b12, Wl, bl):
    ei = edge_index.astype(jnp.int32).reshape(2 * N_EDGES)
    b01r = b01.reshape(1, D)
    b02r = b02.reshape(1, D)
    b11r = b11.reshape(1, D)
    b12r = b12.reshape(1, D)
    blr = bl.reshape(1, D)
    Wl_top = Wl[:D]
    Wl_bot = Wl[D:]

    agg0 = _sc_segment_sum(x, ei)
    p0 = pl.pallas_call(
        _pre_body, grid=_GRID,
        in_specs=[_row_spec(), _full_spec(D, D), _full_spec(1, D)],
        out_specs=_row_spec(), out_shape=_F32,
    )(x, W01, b01r)

    h1 = pl.pallas_call(
        _l0_body, grid=_GRID,
        in_specs=[_row_spec(), _agg_spec(), _full_spec(D, D),
                  _full_spec(D, D), _full_spec(1, D)],
        out_specs=_row_spec(), out_shape=_F32,
    )(p0, agg0, W01, W02, b02r)

    agg1 = _sc_segment_sum(h1, ei)
    p1, part = pl.pallas_call(
        _mid_body, grid=_GRID,
        in_specs=[_row_spec(), _full_spec(D, D), _full_spec(1, D),
                  _full_spec(D, D), _full_spec(1, D)],
        out_specs=[_row_spec(), _row_spec()], out_shape=[_F32, _F32],
    )(h1, W11, b11r, Wl_top, blr)

    out = pl.pallas_call(
        _l1_body, grid=_GRID,
        in_specs=[_row_spec(), _agg_spec(), _full_spec(D, D),
                  _full_spec(D, D), _full_spec(1, D), _full_spec(D, D),
                  _row_spec()],
        out_specs=_row_spec(), out_shape=_F32,
    )(p1, agg1, W11, W12, b12r, Wl_bot, part)
    return out


# speedup vs baseline: 1.0243x; 1.0243x over previous
import jax
import jax.numpy as jnp
from jax import lax
from jax.experimental import pallas as pl
from jax.experimental.pallas import tpu as pltpu
from jax.experimental.pallas import tpu_sc as plsc

N_NODES = 10000
D = 128
N_EDGES = 320000

_NC = 2
_NS = 16
_NW = _NC * _NS
_EPW = N_EDGES // _NW
_K = 80
_NCHUNK = _EPW // _K
_RPT = 624
_TAIL0 = _NS * _RPT
_TAIL = N_NODES - _TAIL0


def _sc_body(h_hbm, ei_hbm, out_hbm, idx_s, idx_d, rows, acc, gsem, ssem):
    c = lax.axis_index("c")
    s = lax.axis_index("s")
    w = s * _NC + c
    row0 = s * _RPT

    isrc = pltpu.async_copy(ei_hbm.at[pl.ds(w * _EPW, _EPW)], idx_s, gsem[0])
    idst = pltpu.async_copy(ei_hbm.at[pl.ds(N_EDGES + w * _EPW, _EPW)],
                            idx_d, gsem[1])

    def _gather(i, b):
        pltpu.async_copy(h_hbm.at[idx_s.at[pl.ds(i * _K, _K)]], rows[b], gsem[b])

    def _gwait(i, b):
        pltpu.make_async_copy(h_hbm.at[idx_s.at[pl.ds(i * _K, _K)]],
                              rows[b], gsem[b]).wait()

    def _scatter(i, b):
        pltpu.async_copy(rows[b], acc.at[idx_d.at[pl.ds(i * _K, _K)]],
                         ssem[b], add=True)

    def _swait(i, b):
        pltpu.make_async_copy(rows[b], acc.at[idx_d.at[pl.ds(i * _K, _K)]],
                              ssem[b]).wait()

    z = jnp.zeros((16,), jnp.float32)

    def _zfill(i, _):
        for j in range(D // 16):
            rows[0][i, pl.ds(j * 16, 16)] = z
        return 0

    lax.fori_loop(0, _K, _zfill, 0)
    isrc.wait()
    idst.wait()
    for t in range(_RPT // _K):
        pltpu.sync_copy(rows[0], acc.at[pl.ds(row0 + t * _K, _K)])
    pltpu.sync_copy(rows[0].at[pl.ds(0, _RPT % _K)],
                    acc.at[pl.ds(row0 + (_RPT // _K) * _K, _RPT % _K)])

    @pl.when(s == 0)
    def _zero_tail():
        pltpu.sync_copy(rows[0].at[pl.ds(0, _TAIL)],
                        acc.at[pl.ds(_TAIL0, _TAIL)])

    plsc.subcore_barrier()

    _gather(0, 0)
    _gather(1, 1)
    _gwait(0, 0)
    _scatter(0, 0)
    _gather(2, 2)
    _gwait(1, 1)
    _scatter(1, 1)
    _swait(0, 0)
    _gather(3, 0)

    def _step(i, b):
        _gwait(i, b)
        _scatter(i, b)
        _swait(i - 1, (b + 2) % 3)
        _gather(i + 2, (b + 2) % 3)

    def _body(j, _):
        i = 3 * j + 2
        _step(i, 2)
        _step(i + 1, 0)
        _step(i + 2, 1)
        return 0

    _L = (_NCHUNK - 4) // 3
    lax.fori_loop(0, _L, _body, 0)
    for i in range(2 + 3 * _L, _NCHUNK):
        _gwait(i, i % 3)
        _scatter(i, i % 3)
        _swait(i - 1, (i - 1) % 3)
        if i + 2 < _NCHUNK:
            _gather(i + 2, (i + 2) % 3)
    _swait(_NCHUNK - 1, (_NCHUNK - 1) % 3)
    plsc.subcore_barrier()
    pltpu.sync_copy(acc.at[pl.ds(row0, _RPT)], out_hbm.at[c, pl.ds(row0, _RPT)])

    @pl.when(s == 0)
    def _copy_tail():
        pltpu.sync_copy(acc.at[pl.ds(_TAIL0, _TAIL)],
                        out_hbm.at[c, pl.ds(_TAIL0, _TAIL)])


def _sc_segment_sum(h, ei):
    mesh = plsc.VectorSubcoreMesh(core_axis_name="c", subcore_axis_name="s")
    f = pl.kernel(
        _sc_body,
        out_type=jax.ShapeDtypeStruct((_NC, N_NODES, D), jnp.float32),
        mesh=mesh,
        scratch_types=[
            pltpu.VMEM((_EPW,), jnp.int32),
            pltpu.VMEM((_EPW,), jnp.int32),
            [pltpu.VMEM((_K, D), jnp.float32) for _ in range(3)],
            pltpu.VMEM_SHARED((N_NODES, D), jnp.float32),
            [pltpu.SemaphoreType.DMA for _ in range(3)],
            [pltpu.SemaphoreType.DMA for _ in range(3)],
        ],
    )
    return f(h, ei)


_BLK = 2000


def _pre_body(x_ref, w_ref, b_ref, p_ref):
    p_ref[...] = jnp.dot(x_ref[...], w_ref[...]) + b_ref[...]


def _mid_body(h_ref, w_ref, b_ref, wl_ref, bl_ref, p_ref, part_ref):
    h = h_ref[...]
    p_ref[...] = jnp.dot(h, w_ref[...]) + b_ref[...]
    part_ref[...] = jnp.dot(h, wl_ref[...]) + bl_ref[...]


def _l0_body(p_ref, a_ref, w1_ref, w2_ref, b2_ref, h_ref):
    t = jnp.maximum(p_ref[...] + jnp.dot(a_ref[0] + a_ref[1], w1_ref[...]), 0.0)
    h_ref[...] = jnp.maximum(jnp.dot(t, w2_ref[...]) + b2_ref[...], 0.0)


def _l1_body(p_ref, a_ref, w1_ref, w2_ref, b2_ref, wl_ref, part_ref, out_ref):
    t = jnp.maximum(p_ref[...] + jnp.dot(a_ref[0] + a_ref[1], w1_ref[...]), 0.0)
    h2 = jnp.maximum(jnp.dot(t, w2_ref[...]) + b2_ref[...], 0.0)
    logits = part_ref[...] + jnp.dot(h2, wl_ref[...])
    m = jnp.max(logits, axis=1, keepdims=True)
    lse = jnp.log(jnp.sum(jnp.exp(logits - m), axis=1, keepdims=True)) + m
    out_ref[...] = logits - lse


def _row_spec():
    return pl.BlockSpec((_BLK, D), lambda i: (i, 0))


def _agg_spec():
    return pl.BlockSpec((_NC, _BLK, D), lambda i: (0, i, 0))


def _full_spec(r, c):
    return pl.BlockSpec((r, c), lambda i: (0, 0))


_GRID = (N_NODES // _BLK,)
_F32 = jax.ShapeDtypeStruct((N_NODES, D), jnp.float32)


def kernel(x, edge_index, W01, b01, W02, b02, W11, b11, W12, b12, Wl, bl):
    ei = edge_index.astype(jnp.int32).reshape(2 * N_EDGES)
    b01r = b01.reshape(1, D)
    b02r = b02.reshape(1, D)
    b11r = b11.reshape(1, D)
    b12r = b12.reshape(1, D)
    blr = bl.reshape(1, D)
    Wl_top = Wl[:D]
    Wl_bot = Wl[D:]

    agg0 = _sc_segment_sum(x, ei)
    p0 = pl.pallas_call(
        _pre_body, grid=_GRID,
        in_specs=[_row_spec(), _full_spec(D, D), _full_spec(1, D)],
        out_specs=_row_spec(), out_shape=_F32,
    )(x, W01, b01r)

    h1 = pl.pallas_call(
        _l0_body, grid=_GRID,
        in_specs=[_row_spec(), _agg_spec(), _full_spec(D, D),
                  _full_spec(D, D), _full_spec(1, D)],
        out_specs=_row_spec(), out_shape=_F32,
    )(p0, agg0, W01, W02, b02r)

    agg1 = _sc_segment_sum(h1, ei)
    p1, part = pl.pallas_call(
        _mid_body, grid=_GRID,
        in_specs=[_row_spec(), _full_spec(D, D), _full_spec(1, D),
                  _full_spec(D, D), _full_spec(1, D)],
        out_specs=[_row_spec(), _row_spec()], out_shape=[_F32, _F32],
    )(h1, W11, b11r, Wl_top, blr)

    out = pl.pallas_call(
        _l1_body, grid=_GRID,
        in_specs=[_row_spec(), _agg_spec(), _full_spec(D, D),
                  _full_spec(D, D), _full_spec(1, D), _full_spec(D, D),
                  _row_spec()],
        out_specs=_row_spec(), out_shape=_F32,
    )(p1, agg1, W11, W12, b12r, Wl_bot, part)
    return out


# speedup vs baseline: 1.0268x; 1.0024x over previous
import jax
import jax.numpy as jnp
from jax import lax
from jax.experimental import pallas as pl
from jax.experimental.pallas import tpu as pltpu
from jax.experimental.pallas import tpu_sc as plsc

N_NODES = 10000
D = 128
N_EDGES = 320000

_NC = 2
_NS = 16
_NW = _NC * _NS
_EPW = N_EDGES // _NW
_K = 80
_NCHUNK = _EPW // _K
_RPT = 624
_TAIL0 = _NS * _RPT
_TAIL = N_NODES - _TAIL0


def _sc_body(h_hbm, ei_hbm, out_hbm, idx_s, idx_d, rows, acc, gsem, ssem):
    c = lax.axis_index("c")
    s = lax.axis_index("s")
    w = s * _NC + c
    row0 = s * _RPT

    isrc = pltpu.async_copy(ei_hbm.at[pl.ds(w * _EPW, _EPW)], idx_s, gsem[0])
    idst = pltpu.async_copy(ei_hbm.at[pl.ds(N_EDGES + w * _EPW, _EPW)],
                            idx_d, gsem[1])

    def _gather(i, b):
        pltpu.async_copy(h_hbm.at[idx_s.at[pl.ds(i * _K, _K)]], rows[b], gsem[b])

    def _gwait(i, b):
        pltpu.make_async_copy(h_hbm.at[idx_s.at[pl.ds(i * _K, _K)]],
                              rows[b], gsem[b]).wait()

    def _scatter(i, b):
        pltpu.async_copy(rows[b], acc.at[idx_d.at[pl.ds(i * _K, _K)]],
                         ssem[b], add=True)

    def _swait(i, b):
        pltpu.make_async_copy(rows[b], acc.at[idx_d.at[pl.ds(i * _K, _K)]],
                              ssem[b]).wait()

    z = jnp.zeros((16,), jnp.float32)

    def _zfill(i, _):
        for j in range(D // 16):
            rows[0][i, pl.ds(j * 16, 16)] = z
        return 0

    lax.fori_loop(0, _K, _zfill, 0)
    isrc.wait()
    idst.wait()
    for t in range(_RPT // _K):
        pltpu.sync_copy(rows[0], acc.at[pl.ds(row0 + t * _K, _K)])
    pltpu.sync_copy(rows[0].at[pl.ds(0, _RPT % _K)],
                    acc.at[pl.ds(row0 + (_RPT // _K) * _K, _RPT % _K)])

    @pl.when(s == 0)
    def _zero_tail():
        pltpu.sync_copy(rows[0].at[pl.ds(0, _TAIL)],
                        acc.at[pl.ds(_TAIL0, _TAIL)])

    plsc.subcore_barrier()

    _gather(0, 0)
    _gather(1, 1)
    _gwait(0, 0)
    _scatter(0, 0)
    _gather(2, 2)
    _gwait(1, 1)
    _scatter(1, 1)
    _swait(0, 0)
    _gather(3, 0)

    def _step(i, b):
        _gwait(i, b)
        _scatter(i, b)
        _swait(i - 1, (b + 2) % 3)
        _gather(i + 2, (b + 2) % 3)

    def _body(j, _):
        i = 3 * j + 2
        _step(i, 2)
        _step(i + 1, 0)
        _step(i + 2, 1)
        return 0

    _L = (_NCHUNK - 4) // 3
    lax.fori_loop(0, _L, _body, 0)
    for i in range(2 + 3 * _L, _NCHUNK):
        _gwait(i, i % 3)
        _scatter(i, i % 3)
        _swait(i - 1, (i - 1) % 3)
        if i + 2 < _NCHUNK:
            _gather(i + 2, (i + 2) % 3)
    _swait(_NCHUNK - 1, (_NCHUNK - 1) % 3)
    plsc.subcore_barrier()
    pltpu.sync_copy(acc.at[pl.ds(row0, _RPT)], out_hbm.at[c, pl.ds(row0, _RPT)])

    @pl.when(s == 0)
    def _copy_tail():
        pltpu.sync_copy(acc.at[pl.ds(_TAIL0, _TAIL)],
                        out_hbm.at[c, pl.ds(_TAIL0, _TAIL)])


def _sc_segment_sum(h, ei):
    mesh = plsc.VectorSubcoreMesh(core_axis_name="c", subcore_axis_name="s")
    f = pl.kernel(
        _sc_body,
        out_type=jax.ShapeDtypeStruct((_NC, N_NODES, D), jnp.float32),
        mesh=mesh,
        scratch_types=[
            pltpu.VMEM((_EPW,), jnp.int32),
            pltpu.VMEM((_EPW,), jnp.int32),
            [pltpu.VMEM((_K, D), jnp.float32) for _ in range(3)],
            pltpu.VMEM_SHARED((N_NODES, D), jnp.float32),
            [pltpu.SemaphoreType.DMA for _ in range(3)],
            [pltpu.SemaphoreType.DMA for _ in range(3)],
        ],
    )
    return f(h, ei)


_BLK = 5000


def _pre_body(x_ref, w_ref, b_ref, p_ref):
    p_ref[...] = jnp.dot(x_ref[...], w_ref[...]) + b_ref[...]


def _mid_body(h_ref, w_ref, b_ref, wl_ref, bl_ref, p_ref, part_ref):
    h = h_ref[...]
    p_ref[...] = jnp.dot(h, w_ref[...]) + b_ref[...]
    part_ref[...] = jnp.dot(h, wl_ref[...]) + bl_ref[...]


def _l0_body(p_ref, a_ref, w1_ref, w2_ref, b2_ref, h_ref):
    t = jnp.maximum(p_ref[...] + jnp.dot(a_ref[0] + a_ref[1], w1_ref[...]), 0.0)
    h_ref[...] = jnp.maximum(jnp.dot(t, w2_ref[...]) + b2_ref[...], 0.0)


def _l1_body(p_ref, a_ref, w1_ref, w2_ref, b2_ref, wl_ref, part_ref, out_ref):
    t = jnp.maximum(p_ref[...] + jnp.dot(a_ref[0] + a_ref[1], w1_ref[...]), 0.0)
    h2 = jnp.maximum(jnp.dot(t, w2_ref[...]) + b2_ref[...], 0.0)
    logits = part_ref[...] + jnp.dot(h2, wl_ref[...])
    m = jnp.max(logits, axis=1, keepdims=True)
    lse = jnp.log(jnp.sum(jnp.exp(logits - m), axis=1, keepdims=True)) + m
    out_ref[...] = logits - lse


def _row_spec():
    return pl.BlockSpec((_BLK, D), lambda i: (i, 0))


def _agg_spec():
    return pl.BlockSpec((_NC, _BLK, D), lambda i: (0, i, 0))


def _full_spec(r, c):
    return pl.BlockSpec((r, c), lambda i: (0, 0))


_GRID = (N_NODES // _BLK,)
_F32 = jax.ShapeDtypeStruct((N_NODES, D), jnp.float32)


def kernel(x, edge_index, W01, b01, W02, b02, W11, b11, W12, b12, Wl, bl):
    ei = edge_index.astype(jnp.int32).reshape(2 * N_EDGES)
    b01r = b01.reshape(1, D)
    b02r = b02.reshape(1, D)
    b11r = b11.reshape(1, D)
    b12r = b12.reshape(1, D)
    blr = bl.reshape(1, D)
    Wl_top = Wl[:D]
    Wl_bot = Wl[D:]

    agg0 = _sc_segment_sum(x, ei)
    p0 = pl.pallas_call(
        _pre_body, grid=_GRID,
        in_specs=[_row_spec(), _full_spec(D, D), _full_spec(1, D)],
        out_specs=_row_spec(), out_shape=_F32,
    )(x, W01, b01r)

    h1 = pl.pallas_call(
        _l0_body, grid=_GRID,
        in_specs=[_row_spec(), _agg_spec(), _full_spec(D, D),
                  _full_spec(D, D), _full_spec(1, D)],
        out_specs=_row_spec(), out_shape=_F32,
    )(p0, agg0, W01, W02, b02r)

    agg1 = _sc_segment_sum(h1, ei)
    p1, part = pl.pallas_call(
        _mid_body, grid=_GRID,
        in_specs=[_row_spec(), _full_spec(D, D), _full_spec(1, D),
                  _full_spec(D, D), _full_spec(1, D)],
        out_specs=[_row_spec(), _row_spec()], out_shape=[_F32, _F32],
    )(h1, W11, b11r, Wl_top, blr)

    out = pl.pallas_call(
        _l1_body, grid=_GRID,
        in_specs=[_row_spec(), _agg_spec(), _full_spec(D, D),
                  _full_spec(D, D), _full_spec(1, D), _full_spec(D, D),
                  _row_spec()],
        out_specs=_row_spec(), out_shape=_F32,
    )(p1, agg1, W11, W12, b12r, Wl_bot, part)
    return out
